# baseline (speedup 1.0000x reference)
import functools

import jax
import jax.numpy as jnp
from jax import lax
from jax.experimental import pallas as pl
from jax.experimental.pallas import tpu as pltpu
from jax.experimental.pallas import tpu_sc as plsc

_N = 10000
_NPAD = 10240
_E = 320000
_D = 128
_DOUT = 16

_NC = 2
_NS = 16
_NTILES = _NC * _NS
_EPT = 10240
_EPAD = _NTILES * _EPT
_CHUNK = 128
_NCH = _EPT // _CHUNK
_ROWS_PS = _NPAD // _NS

_BR = 1000
_NB = _N // _BR


def _sc_mesh():
    return plsc.VectorSubcoreMesh(core_axis_name="c", subcore_axis_name="s")


def _zero_init(buf, accum, s):
    @pl.loop(0, _CHUNK)
    def _(r):
        @pl.loop(0, _D // 16)
        def _(q):
            buf[r, pl.ds(q * 16, 16)] = jnp.zeros((16,), jnp.float32)

    @pl.loop(0, _ROWS_PS // _CHUNK)
    def _(i):
        pltpu.sync_copy(buf, accum.at[pl.ds(s * _ROWS_PS + i * _CHUNK, _CHUNK)])


def _drain(accum, out0, out1, c, s):
    @pl.when(c == 0)
    def _():
        pltpu.sync_copy(accum.at[pl.ds(s * _ROWS_PS, _ROWS_PS)],
                        out0.at[pl.ds(s * _ROWS_PS, _ROWS_PS)])

    @pl.when(c == 1)
    def _():
        pltpu.sync_copy(accum.at[pl.ds(s * _ROWS_PS, _ROWS_PS)],
                        out1.at[pl.ds(s * _ROWS_PS, _ROWS_PS)])


_PART = (jax.ShapeDtypeStruct((_NPAD, _D), jnp.float32),) * _NC


def _sc_degree(dst2d):
    @functools.partial(
        pl.kernel,
        mesh=_sc_mesh(),
        out_type=_PART,
        scratch_types=[
            pltpu.VMEM((_NCH, _CHUNK), jnp.int32),
            pltpu.VMEM((_CHUNK, _D), jnp.float32),
            pltpu.VMEM_SHARED((_NPAD, _D), jnp.float32),
            pltpu.SemaphoreType.DMA,
        ],
    )
    def k(dst_hbm, out0, out1, didx, ones_v, accum, sem):
        c = lax.axis_index("c")
        s = lax.axis_index("s")
        t = c * _NS + s
        _zero_init(ones_v, accum, s)

        @pl.loop(0, _CHUNK)
        def _(r):
            @pl.loop(0, _D // 16)
            def _(q):
                ones_v[r, pl.ds(q * 16, 16)] = jnp.ones((16,), jnp.float32)

        pltpu.sync_copy(dst_hbm.at[pl.ds(t * _NCH, _NCH)], didx)
        plsc.subcore_barrier()

        depth = 8

        @pl.loop(0, depth)
        def _(j):
            pltpu.async_copy(ones_v, accum.at[didx.at[j]], sem, add=True)

        @pl.loop(depth, _NCH)
        def _(j):
            pltpu.make_async_copy(ones_v, accum.at[didx.at[j]], sem).wait()
            pltpu.async_copy(ones_v, accum.at[didx.at[j]], sem, add=True)

        @pl.loop(0, depth)
        def _(j):
            pltpu.make_async_copy(ones_v, accum.at[didx.at[j]], sem).wait()

        plsc.subcore_barrier()
        _drain(accum, out0, out1, c, s)

    return k(dst2d)


_NBUF = 2
_NHALF = 2
_HCH = _NCH // _NHALF
_NGRP = _HCH // _NBUF


def _sc_aggregate(y, src2d, dst2d):
    @functools.partial(
        pl.kernel,
        mesh=_sc_mesh(),
        out_type=_PART,
        scratch_types=[
            pltpu.VMEM((_HCH, _CHUNK), jnp.int32),
            pltpu.VMEM((_HCH, _CHUNK), jnp.int32),
            pltpu.VMEM((_NBUF, _CHUNK, _D), jnp.float32),
            pltpu.VMEM_SHARED((_NPAD, _D), jnp.float32),
            pltpu.SemaphoreType.DMA((_NBUF,)),
            pltpu.SemaphoreType.DMA((_NBUF,)),
        ],
    )
    def k(y_hbm, src_hbm, dst_hbm, out0, out1, sidx, didx, rows, accum,
          gsem, ssem):
        c = lax.axis_index("c")
        s = lax.axis_index("s")
        t = c * _NS + s
        _zero_init(rows.at[0], accum, s)
        plsc.subcore_barrier()

        for h in range(_NHALF):
            pltpu.sync_copy(src_hbm.at[pl.ds(t * _NCH + h * _HCH, _HCH)], sidx)
            pltpu.sync_copy(dst_hbm.at[pl.ds(t * _NCH + h * _HCH, _HCH)], didx)

            @pl.loop(0, _NGRP)
            def _(g):
                base = g * _NBUF
                for b in range(_NBUF):
                    @pl.when(g > 0)
                    def _():
                        pltpu.make_async_copy(
                            rows.at[b], accum.at[didx.at[base + b]],
                            ssem.at[b]).wait()

                    pltpu.async_copy(y_hbm.at[sidx.at[base + b]], rows.at[b],
                                     gsem.at[b])
                for b in range(_NBUF):
                    pltpu.make_async_copy(y_hbm.at[sidx.at[base + b]],
                                          rows.at[b], gsem.at[b]).wait()
                    pltpu.async_copy(rows.at[b], accum.at[didx.at[base + b]],
                                     ssem.at[b], add=True)

            for b in range(_NBUF):
                pltpu.make_async_copy(rows.at[b], accum.at[didx.at[b]],
                                      ssem.at[b]).wait()

        plsc.subcore_barrier()
        _drain(accum, out0, out1, c, s)

    return k(y, src2d, dst2d)


def _dinv_block(p0, p1):
    return lax.rsqrt(p0[:, 0:1] + p1[:, 0:1] + 1.0)


def _row_spec(width=_D):
    return pl.BlockSpec((_BR, width), lambda i: (i, 0))


def _full(shape):
    return pl.BlockSpec(shape, lambda i: (0,) * len(shape))


def _tc_first(x, W1, degp):
    def body(x_ref, w_ref, p0_ref, p1_ref, y_ref, d_ref):
        dinv = _dinv_block(p0_ref, p1_ref)
        d_ref[...] = jnp.broadcast_to(dinv, (_BR, 16))
        y_ref[...] = dinv * jnp.dot(x_ref[...], w_ref[...],
                                    preferred_element_type=jnp.float32)

    return pl.pallas_call(
        body,
        grid=(_NB,),
        in_specs=[_row_spec(), _full((_D, _D)), _row_spec(), _row_spec()],
        out_specs=(_row_spec(), pl.BlockSpec((_BR, 16), lambda i: (i, 0))),
        out_shape=(jax.ShapeDtypeStruct((_N, _D), jnp.float32),
                   jax.ShapeDtypeStruct((_N, 16), jnp.float32)),
    )(x, W1, degp[0], degp[1])


def _tc_mid(aggp, y, dinv16, b, W, mask=None):
    def body(*refs):
        if mask is None:
            a0, a1, y_ref, d_ref, b_ref, w_ref, o_ref = refs
        else:
            a0, a1, y_ref, d_ref, b_ref, w_ref, m_ref, o_ref = refs
        dinv = d_ref[:, 0:1]
        h = jnp.tanh(dinv * (a0[...] + a1[...] + y_ref[...]) + b_ref[...])
        if mask is not None:
            h = h * m_ref[...]
        o_ref[...] = dinv * jnp.dot(h, w_ref[...],
                                    preferred_element_type=jnp.float32)

    dspec = pl.BlockSpec((_BR, 16), lambda i: (i, 0))
    in_specs = [_row_spec(), _row_spec(), _row_spec(), dspec, _full((1, _D)),
                _full((_D, _D))]
    args = [aggp[0], aggp[1], y, dinv16, b.reshape(1, _D), W]
    if mask is not None:
        in_specs.append(_row_spec())
        args.append(mask)
    return pl.pallas_call(
        body,
        grid=(_NB,),
        in_specs=in_specs,
        out_specs=_row_spec(),
        out_shape=jax.ShapeDtypeStruct((_N, _D), jnp.float32),
    )(*args)


def _tc_last(aggp, y, dinv16, b3, Wc, bc):
    def body(a0, a1, y_ref, d_ref, b_ref, wc_ref, bc_ref, h_ref, o_ref):
        dinv = d_ref[:, 0:1]
        h = jnp.tanh(dinv * (a0[...] + a1[...] + y_ref[...]) + b_ref[...])
        h_ref[...] = h
        o_ref[...] = jnp.dot(h, wc_ref[...],
                             preferred_element_type=jnp.float32) + bc_ref[...]

    dspec = pl.BlockSpec((_BR, 16), lambda i: (i, 0))
    return pl.pallas_call(
        body,
        grid=(_NB,),
        in_specs=[_row_spec(), _row_spec(), _row_spec(), dspec, _full((1, _D)),
                  _full((_D, _DOUT)), _full((1, _DOUT))],
        out_specs=(pl.BlockSpec((_BR, _D), lambda i: (i, 0)),
                   pl.BlockSpec((_BR, _DOUT), lambda i: (i, 0))),
        out_shape=(jax.ShapeDtypeStruct((_N, _D), jnp.float32),
                   jax.ShapeDtypeStruct((_N, _DOUT), jnp.float32)),
    )(aggp[0], aggp[1], y, dinv16, b3.reshape(1, _D), Wc, bc.reshape(1, _DOUT))


def kernel(x, edge_index, W1, b1, W2, b2, W3, b3, Wc, bc):
    f32 = jnp.float32
    npads = _EPAD - _E
    padidx = _N + (jnp.arange(npads, dtype=jnp.int32) % (_NPAD - _N))
    src2d = jnp.concatenate(
        [edge_index[0], jnp.zeros((npads,), jnp.int32)]).reshape(-1, _CHUNK)
    dst2d = jnp.concatenate([edge_index[1], padidx]).reshape(-1, _CHUNK)
    keep = jax.random.bernoulli(jax.random.key(42), 0.8, (_N, _D))
    mask = keep.astype(f32) / 0.8

    degp = _sc_degree(dst2d)

    y1, dinv16 = _tc_first(x, W1, degp)
    agg1 = _sc_aggregate(y1, src2d, dst2d)
    y2 = _tc_mid(agg1, y1, dinv16, b1, W2, mask=mask)

    agg2 = _sc_aggregate(y2, src2d, dst2d)
    y3 = _tc_mid(agg2, y2, dinv16, b2, W3)

    agg3 = _sc_aggregate(y3, src2d, dst2d)
    h3, out = _tc_last(agg3, y3, dinv16, b3, Wc, bc)

    return out, h3


# speedup vs baseline: 2.7650x; 2.7650x over previous
import functools

import jax
import jax.numpy as jnp
from jax import lax
from jax.experimental import pallas as pl
from jax.experimental.pallas import tpu as pltpu
from jax.experimental.pallas import tpu_sc as plsc

_N = 10000
_NPAD = 10240
_E = 320000
_D = 128
_DOUT = 16

_NC = 2
_NS = 16
_NTILES = _NC * _NS
_EPT = 10240
_EPAD = _NTILES * _EPT
_CHUNK = 128
_NCH = _EPT // _CHUNK
_ROWS_PS = _NPAD // _NS

_BR = 1000
_NB = _N // _BR


def _sc_mesh():
    return plsc.VectorSubcoreMesh(core_axis_name="c", subcore_axis_name="s")


def _zero_init(buf, accum, s):
    @pl.loop(0, _CHUNK)
    def _(r):
        @pl.loop(0, _D // 16)
        def _(q):
            buf[r, pl.ds(q * 16, 16)] = jnp.zeros((16,), jnp.float32)

    @pl.loop(0, _ROWS_PS // _CHUNK)
    def _(i):
        pltpu.sync_copy(buf, accum.at[pl.ds(s * _ROWS_PS + i * _CHUNK, _CHUNK)])


def _drain(accum, out0, out1, c, s):
    @pl.when(c == 0)
    def _():
        pltpu.sync_copy(accum.at[pl.ds(s * _ROWS_PS, _ROWS_PS)],
                        out0.at[pl.ds(s * _ROWS_PS, _ROWS_PS)])

    @pl.when(c == 1)
    def _():
        pltpu.sync_copy(accum.at[pl.ds(s * _ROWS_PS, _ROWS_PS)],
                        out1.at[pl.ds(s * _ROWS_PS, _ROWS_PS)])


_PART = (jax.ShapeDtypeStruct((_NPAD, _D), jnp.float32),) * _NC


def _sc_degree(dst2d):
    @functools.partial(
        pl.kernel,
        mesh=_sc_mesh(),
        out_type=_PART,
        scratch_types=[
            pltpu.VMEM((_NCH, _CHUNK), jnp.int32),
            pltpu.VMEM((_CHUNK, _D), jnp.float32),
            pltpu.VMEM_SHARED((_NPAD, _D), jnp.float32),
            pltpu.SemaphoreType.DMA,
        ],
    )
    def k(dst_hbm, out0, out1, didx, ones_v, accum, sem):
        c = lax.axis_index("c")
        s = lax.axis_index("s")
        t = c * _NS + s
        _zero_init(ones_v, accum, s)

        @pl.loop(0, _CHUNK)
        def _(r):
            @pl.loop(0, _D // 16)
            def _(q):
                ones_v[r, pl.ds(q * 16, 16)] = jnp.ones((16,), jnp.float32)

        pltpu.sync_copy(dst_hbm.at[pl.ds(t * _NCH, _NCH)], didx)
        plsc.subcore_barrier()

        depth = 8

        @pl.loop(0, depth)
        def _(j):
            pltpu.async_copy(ones_v, accum.at[didx.at[j]], sem, add=True)

        @pl.loop(depth, _NCH)
        def _(j):
            pltpu.make_async_copy(ones_v, accum.at[didx.at[j]], sem).wait()
            pltpu.async_copy(ones_v, accum.at[didx.at[j]], sem, add=True)

        @pl.loop(0, depth)
        def _(j):
            pltpu.make_async_copy(ones_v, accum.at[didx.at[j]], sem).wait()

        plsc.subcore_barrier()
        _drain(accum, out0, out1, c, s)

    return k(dst2d)


_NBUF = 2
_NHALF = 2
_HCH = _NCH // _NHALF
_NGRP = _HCH // _NBUF


def _sc_aggregate(y, src2d, dst2d):
    @functools.partial(
        pl.kernel,
        mesh=_sc_mesh(),
        out_type=_PART,
        scratch_types=[
            pltpu.VMEM((_HCH, _CHUNK), jnp.int32),
            pltpu.VMEM((_HCH, _CHUNK), jnp.int32),
            pltpu.VMEM((_NBUF, _CHUNK, _D), jnp.float32),
            pltpu.VMEM_SHARED((_NPAD, _D), jnp.float32),
            pltpu.SemaphoreType.DMA((_NBUF,)),
            pltpu.SemaphoreType.DMA((_NBUF,)),
        ],
    )
    def k(y_hbm, src_hbm, dst_hbm, out0, out1, sidx, didx, rows, accum,
          gsem, ssem):
        c = lax.axis_index("c")
        s = lax.axis_index("s")
        t = c * _NS + s
        _zero_init(rows.at[0], accum, s)
        plsc.subcore_barrier()

        for h in range(_NHALF):
            pltpu.sync_copy(src_hbm.at[pl.ds(t * _NCH + h * _HCH, _HCH)], sidx)
            pltpu.sync_copy(dst_hbm.at[pl.ds(t * _NCH + h * _HCH, _HCH)], didx)

            @pl.loop(0, _NGRP)
            def _(g):
                base = g * _NBUF
                for b in range(_NBUF):
                    @pl.when(g > 0)
                    def _():
                        pltpu.make_async_copy(
                            rows.at[b], accum.at[didx.at[base + b]],
                            ssem.at[b]).wait()

                    pltpu.async_copy(y_hbm.at[sidx.at[base + b]], rows.at[b],
                                     gsem.at[b])
                for b in range(_NBUF):
                    pltpu.make_async_copy(y_hbm.at[sidx.at[base + b]],
                                          rows.at[b], gsem.at[b]).wait()
                    pltpu.async_copy(rows.at[b], accum.at[didx.at[base + b]],
                                     ssem.at[b], add=True)

            for b in range(_NBUF):
                pltpu.make_async_copy(rows.at[b], accum.at[didx.at[b]],
                                      ssem.at[b]).wait()

        plsc.subcore_barrier()
        _drain(accum, out0, out1, c, s)

    return k(y, src2d, dst2d)


def _dinv_block(p0, p1):
    return lax.rsqrt(p0[:, 0:1] + p1[:, 0:1] + 1.0)


def _row_spec(width=_D):
    return pl.BlockSpec((_BR, width), lambda i: (i, 0))


def _full(shape):
    return pl.BlockSpec(shape, lambda i: (0,) * len(shape))


def _tc_first(x, W1, degp):
    def body(x_ref, w_ref, p0_ref, p1_ref, y_ref, d_ref):
        dinv = _dinv_block(p0_ref, p1_ref)
        d_ref[...] = jnp.broadcast_to(dinv, (_BR, 16))
        y_ref[...] = dinv * jnp.dot(x_ref[...], w_ref[...],
                                    preferred_element_type=jnp.float32)

    return pl.pallas_call(
        body,
        grid=(_NB,),
        in_specs=[_row_spec(), _full((_D, _D)), _row_spec(), _row_spec()],
        out_specs=(_row_spec(), pl.BlockSpec((_BR, 16), lambda i: (i, 0))),
        out_shape=(jax.ShapeDtypeStruct((_N, _D), jnp.float32),
                   jax.ShapeDtypeStruct((_N, 16), jnp.float32)),
    )(x, W1, degp[0], degp[1])


def _tc_mid(aggp, y, dinv16, b, W, mask=None):
    def body(*refs):
        if mask is None:
            a0, a1, y_ref, d_ref, b_ref, w_ref, o_ref = refs
        else:
            a0, a1, y_ref, d_ref, b_ref, w_ref, m_ref, o_ref = refs
        dinv = d_ref[:, 0:1]
        h = jnp.tanh(dinv * (a0[...] + a1[...] + y_ref[...]) + b_ref[...])
        if mask is not None:
            h = h * m_ref[...]
        o_ref[...] = dinv * jnp.dot(h, w_ref[...],
                                    preferred_element_type=jnp.float32)

    dspec = pl.BlockSpec((_BR, 16), lambda i: (i, 0))
    in_specs = [_row_spec(), _row_spec(), _row_spec(), dspec, _full((1, _D)),
                _full((_D, _D))]
    args = [aggp[0], aggp[1], y, dinv16, b.reshape(1, _D), W]
    if mask is not None:
        in_specs.append(_row_spec())
        args.append(mask)
    return pl.pallas_call(
        body,
        grid=(_NB,),
        in_specs=in_specs,
        out_specs=_row_spec(),
        out_shape=jax.ShapeDtypeStruct((_N, _D), jnp.float32),
    )(*args)


def _tc_last(aggp, y, dinv16, b3, Wc, bc):
    def body(a0, a1, y_ref, d_ref, b_ref, wc_ref, bc_ref, h_ref, o_ref):
        dinv = d_ref[:, 0:1]
        h = jnp.tanh(dinv * (a0[...] + a1[...] + y_ref[...]) + b_ref[...])
        h_ref[...] = h
        o_ref[...] = jnp.dot(h, wc_ref[...],
                             preferred_element_type=jnp.float32) + bc_ref[...]

    dspec = pl.BlockSpec((_BR, 16), lambda i: (i, 0))
    return pl.pallas_call(
        body,
        grid=(_NB,),
        in_specs=[_row_spec(), _row_spec(), _row_spec(), dspec, _full((1, _D)),
                  _full((_D, _DOUT)), _full((1, _DOUT))],
        out_specs=(pl.BlockSpec((_BR, _D), lambda i: (i, 0)),
                   pl.BlockSpec((_BR, _DOUT), lambda i: (i, 0))),
        out_shape=(jax.ShapeDtypeStruct((_N, _D), jnp.float32),
                   jax.ShapeDtypeStruct((_N, _DOUT), jnp.float32)),
    )(aggp[0], aggp[1], y, dinv16, b3.reshape(1, _D), Wc, bc.reshape(1, _DOUT))


def kernel(x, edge_index, W1, b1, W2, b2, W3, b3, Wc, bc):
    f32 = jnp.float32
    npads = _EPAD - _E
    padidx = _N + (jnp.arange(npads, dtype=jnp.int32) % (_NPAD - _N))
    padsrc = jnp.arange(npads, dtype=jnp.int32) % _N
    src2d = jnp.concatenate([edge_index[0], padsrc]).reshape(-1, _CHUNK)
    dst2d = jnp.concatenate([edge_index[1], padidx]).reshape(-1, _CHUNK)
    keep = jax.random.bernoulli(jax.random.key(42), 0.8, (_N, _D))
    mask = keep.astype(f32) / 0.8

    degp = _sc_degree(dst2d)

    y1, dinv16 = _tc_first(x, W1, degp)
    agg1 = _sc_aggregate(y1, src2d, dst2d)
    y2 = _tc_mid(agg1, y1, dinv16, b1, W2, mask=mask)

    agg2 = _sc_aggregate(y2, src2d, dst2d)
    y3 = _tc_mid(agg2, y2, dinv16, b2, W3)

    agg3 = _sc_aggregate(y3, src2d, dst2d)
    h3, out = _tc_last(agg3, y3, dinv16, b3, Wc, bc)

    return out, h3
